# baseline (speedup 1.0000x reference)
import functools
import jax
import jax.numpy as jnp
from jax import lax
from jax.experimental import pallas as pl
from jax.experimental.pallas import tpu as pltpu
from jax.experimental.pallas import tpu_sc as plsc

NC, NS = 2, 16
NW = NC * NS
CS = 32
UNROLL = 8


def kernel(x, pe):
    B, S, D = x.shape
    xf = x.reshape(B * S * D)
    pef = pe.reshape(-1)
    s_per_w = S // NW
    n_sub = s_per_w // CS
    chunk = CS * D
    n_add = chunk // (16 * UNROLL)

    mesh = plsc.VectorSubcoreMesh(core_axis_name="c", subcore_axis_name="s")

    @functools.partial(
        pl.kernel,
        mesh=mesh,
        out_type=jax.ShapeDtypeStruct((B * S * D,), jnp.float32),
        scratch_types=[
            pltpu.VMEM((chunk,), jnp.float32),
            pltpu.VMEM((chunk,), jnp.float32),
        ],
    )
    def k(x_hbm, pe_hbm, out_hbm, bufx, bufp):
        wid = lax.axis_index("s") * NC + lax.axis_index("c")
        s_base = wid * s_per_w
        for j in range(n_sub):
            p_off = (s_base + j * CS) * D
            pltpu.sync_copy(pe_hbm.at[pl.ds(p_off, chunk)], bufp)
            for b in range(B):
                x_off = (b * S + s_base + j * CS) * D
                pltpu.sync_copy(x_hbm.at[pl.ds(x_off, chunk)], bufx)

                def add_body(i):
                    base = i * (16 * UNROLL)
                    for u in range(UNROLL):
                        sl = pl.ds(base + u * 16, 16)
                        bufx[sl] = bufx[sl] + bufp[sl]

                lax.fori_loop(0, n_add, lambda i, c: (add_body(i), c)[1], 0)
                pltpu.sync_copy(bufx, out_hbm.at[pl.ds(x_off, chunk)])

    out = k(xf, pef)
    return out.reshape(B, S, D)


# speedup vs baseline: 4.2807x; 4.2807x over previous
import jax
import jax.numpy as jnp
from jax.experimental import pallas as pl

BLK = 1024


def _add_pe_kernel(x_ref, pe_ref, o_ref):
    o_ref[...] = x_ref[...] + pe_ref[...]


def kernel(x, pe):
    B, S, D = x.shape
    xf = x.reshape(B * S, D)
    nS = S // BLK
    grid = (B * S // BLK,)
    out = pl.pallas_call(
        _add_pe_kernel,
        grid=grid,
        in_specs=[
            pl.BlockSpec((BLK, D), lambda i: (i, 0)),
            pl.BlockSpec((BLK, D), lambda i: (i % nS, 0)),
        ],
        out_specs=pl.BlockSpec((BLK, D), lambda i: (i, 0)),
        out_shape=jax.ShapeDtypeStruct((B * S, D), x.dtype),
    )(xf, pe)
    return out.reshape(B, S, D)
